# baseline (speedup 1.0000x reference)
import functools

import jax
import jax.numpy as jnp
from jax import lax
from jax.experimental import pallas as pl
from jax.experimental.pallas import tpu as pltpu
from jax.experimental.pallas import tpu_sc as plsc

_BATCH = 16384
_EMBED = 16
_NC = 2
_NS = 16
_NW = _NC * _NS
_B_PER_W = _BATCH // _NW
_W_PER_W = _B_PER_W * _EMBED


@jax.jit
def _context_embedding_sc(ht_flat, pt_flat, hw_idx, pw_idx, dh_idx, dp_idx):
    mesh = plsc.VectorSubcoreMesh(core_axis_name="c", subcore_axis_name="s")

    @functools.partial(
        pl.kernel,
        mesh=mesh,
        out_type=jax.ShapeDtypeStruct((_BATCH * 2 * _EMBED,), jnp.float32),
        scratch_types=[
            pltpu.VMEM((_W_PER_W,), jnp.int32),
            pltpu.VMEM((_W_PER_W,), jnp.int32),
            pltpu.VMEM((_W_PER_W,), jnp.int32),
            pltpu.VMEM((_W_PER_W,), jnp.int32),
            pltpu.VMEM((_W_PER_W,), jnp.float32),
            pltpu.VMEM((_W_PER_W,), jnp.float32),
            pltpu.SemaphoreType.DMA,
            pltpu.SemaphoreType.DMA,
        ],
    )
    def k(ht_hbm, pt_hbm, hw_hbm, pw_hbm, dh_hbm, dp_hbm, out_hbm,
          hw_v, pw_v, dh_v, dp_v, hwords_v, pwords_v, sem_h, sem_p):
        wid = lax.axis_index("s") * _NC + lax.axis_index("c")
        base = wid * _W_PER_W
        pltpu.sync_copy(hw_hbm.at[pl.ds(base, _W_PER_W)], hw_v)
        pltpu.sync_copy(pw_hbm.at[pl.ds(base, _W_PER_W)], pw_v)
        pltpu.sync_copy(dh_hbm.at[pl.ds(base, _W_PER_W)], dh_v)
        pltpu.sync_copy(dp_hbm.at[pl.ds(base, _W_PER_W)], dp_v)
        gh = pltpu.async_copy(ht_hbm.at[hw_v], hwords_v, sem_h)
        gp = pltpu.async_copy(pt_hbm.at[pw_v], pwords_v, sem_p)
        gh.wait()
        gp.wait()
        sh = pltpu.async_copy(hwords_v, out_hbm.at[dh_v], sem_h)
        sp = pltpu.async_copy(pwords_v, out_hbm.at[dp_v], sem_p)
        sh.wait()
        sp.wait()

    return k(ht_flat, pt_flat, hw_idx, pw_idx, dh_idx, dp_idx)


def kernel(hour_idx, phone_idx, hour_table, phone_table):
    sub = jnp.arange(_EMBED, dtype=jnp.int32)
    hw_idx = (hour_idx.astype(jnp.int32)[:, None] * _EMBED + sub).reshape(-1)
    pw_idx = (phone_idx.astype(jnp.int32)[:, None] * _EMBED + sub).reshape(-1)
    row = jnp.arange(_BATCH, dtype=jnp.int32)[:, None] * (2 * _EMBED)
    dh_idx = (row + sub).reshape(-1)
    dp_idx = (row + _EMBED + sub).reshape(-1)
    out_flat = _context_embedding_sc(
        hour_table.reshape(-1),
        phone_table.reshape(-1),
        hw_idx, pw_idx, dh_idx, dp_idx,
    )
    return out_flat.reshape(_BATCH, 2 * _EMBED)


# speedup vs baseline: 20.5391x; 20.5391x over previous
import functools

import jax
import jax.numpy as jnp
from jax import lax
from jax.experimental import pallas as pl
from jax.experimental.pallas import tpu as pltpu
from jax.experimental.pallas import tpu_sc as plsc

_BATCH = 16384
_EMBED = 16
_HOUR_VOCAB = 24
_NC = 2
_NS = 16
_NW = _NC * _NS
_B_PER_W = _BATCH // _NW


@jax.jit
def _context_embedding_sc(ht_flat, pt_wide, hoff, psup, poff):
    mesh = plsc.VectorSubcoreMesh(core_axis_name="c", subcore_axis_name="s")

    @functools.partial(
        pl.kernel,
        mesh=mesh,
        out_type=jax.ShapeDtypeStruct((_BATCH * 2 * _EMBED,), jnp.float32),
        scratch_types=[
            pltpu.VMEM((_HOUR_VOCAB * _EMBED,), jnp.float32),
            pltpu.VMEM((_B_PER_W,), jnp.int32),
            pltpu.VMEM((_B_PER_W,), jnp.int32),
            pltpu.VMEM((_B_PER_W,), jnp.int32),
            pltpu.VMEM((_B_PER_W, 128), jnp.float32),
            pltpu.VMEM((_B_PER_W * 2 * _EMBED,), jnp.float32),
            pltpu.SemaphoreType.DMA,
        ],
    )
    def k(ht_hbm, pt_hbm, hoff_hbm, psup_hbm, poff_hbm, out_hbm,
          ht_v, hoff_v, psup_v, poff_v, prows_v, cat_v, sem):
        wid = lax.axis_index("s") * _NC + lax.axis_index("c")
        base = wid * _B_PER_W
        pltpu.sync_copy(ht_hbm, ht_v)
        pltpu.sync_copy(hoff_hbm.at[pl.ds(base, _B_PER_W)], hoff_v)
        pltpu.sync_copy(psup_hbm.at[pl.ds(base, _B_PER_W)], psup_v)
        pltpu.sync_copy(poff_hbm.at[pl.ds(base, _B_PER_W)], poff_v)
        gp = pltpu.async_copy(pt_hbm.at[psup_v], prows_v, sem)

        @pl.loop(0, _B_PER_W // 16)
        def _(g):
            hvec = hoff_v[pl.ds(g * 16, 16)]
            for j in range(16):
                i = g * 16 + j
                cat_v.at[pl.ds(i * 32, _EMBED)][...] = (
                    ht_v.at[pl.ds(hvec[j], _EMBED)][...])

        gp.wait()

        @pl.loop(0, _B_PER_W // 16)
        def _(g):
            pvec = poff_v[pl.ds(g * 16, 16)]
            for j in range(16):
                i = g * 16 + j
                cat_v.at[pl.ds(i * 32 + _EMBED, _EMBED)][...] = (
                    prows_v.at[i, pl.ds(pvec[j], _EMBED)][...])

        pltpu.sync_copy(cat_v,
                        out_hbm.at[pl.ds(base * 2 * _EMBED,
                                         _B_PER_W * 2 * _EMBED)])

    return k(ht_flat, pt_wide, hoff, psup, poff)


def kernel(hour_idx, phone_idx, hour_table, phone_table):
    hi = hour_idx.astype(jnp.int32)
    pi = phone_idx.astype(jnp.int32)
    out_flat = _context_embedding_sc(
        hour_table.reshape(-1),
        phone_table.reshape(-1, 128),
        hi * _EMBED,
        pi // 8,
        (pi % 8) * _EMBED,
    )
    return out_flat.reshape(_BATCH, 2 * _EMBED)


# speedup vs baseline: 20.7362x; 1.0096x over previous
import functools

import jax
import jax.numpy as jnp
from jax import lax
from jax.experimental import pallas as pl
from jax.experimental.pallas import tpu as pltpu
from jax.experimental.pallas import tpu_sc as plsc

_BATCH = 16384
_EMBED = 16
_HOUR_VOCAB = 24
_PHONE_VOCAB = 100000
_NC = 2
_NS = 16
_NW = _NC * _NS
_B_PER_W = _BATCH // _NW
_G = 16


@jax.jit
def _context_embedding_sc(hour_idx, phone_idx, hour_table, pt_wide):
    mesh = plsc.VectorSubcoreMesh(core_axis_name="c", subcore_axis_name="s")

    @functools.partial(
        pl.kernel,
        mesh=mesh,
        out_type=jax.ShapeDtypeStruct((_BATCH * 2 * _EMBED,), jnp.float32),
        scratch_types=[
            pltpu.VMEM((_HOUR_VOCAB * _EMBED,), jnp.float32),
            pltpu.VMEM((_B_PER_W,), jnp.int32),
            pltpu.VMEM((_B_PER_W,), jnp.int32),
            pltpu.VMEM((_B_PER_W,), jnp.int32),
            pltpu.VMEM((_B_PER_W, 128), jnp.float32),
            pltpu.VMEM((_B_PER_W * 2 * _EMBED,), jnp.float32),
            pltpu.SemaphoreType.DMA,
        ],
    )
    def k(hi_hbm, pi_hbm, ht_hbm, pt_hbm, out_hbm,
          ht_v, hi_v, pi_v, psup_v, prows_v, cat_v, sem):
        wid = lax.axis_index("s") * _NC + lax.axis_index("c")
        base = wid * _B_PER_W
        pltpu.sync_copy(hi_hbm.at[pl.ds(base, _B_PER_W)], hi_v)
        pltpu.sync_copy(pi_hbm.at[pl.ds(base, _B_PER_W)], pi_v)
        pltpu.sync_copy(ht_hbm, ht_v)

        @pl.loop(0, _B_PER_W // _G)
        def _(g):
            psup_v.at[pl.ds(g * _G, _G)][...] = (
                pi_v[pl.ds(g * _G, _G)] >> 3)

        gp = pltpu.async_copy(pt_hbm.at[psup_v], prows_v, sem)

        @pl.loop(0, _B_PER_W // _G)
        def _(g):
            hvec = hi_v[pl.ds(g * _G, _G)] * _EMBED
            for j in range(_G):
                i = g * _G + j
                cat_v.at[pl.ds(i * 32, _EMBED)][...] = (
                    ht_v.at[pl.ds(hvec[j], _EMBED)][...])

        gp.wait()

        @pl.loop(0, _B_PER_W // _G)
        def _(g):
            pvec = (pi_v[pl.ds(g * _G, _G)] & 7) * _EMBED
            for j in range(_G):
                i = g * _G + j
                cat_v.at[pl.ds(i * 32 + _EMBED, _EMBED)][...] = (
                    prows_v.at[i, pl.ds(pvec[j], _EMBED)][...])

        pltpu.sync_copy(cat_v,
                        out_hbm.at[pl.ds(base * 2 * _EMBED,
                                         _B_PER_W * 2 * _EMBED)])

    return k(hour_idx, phone_idx, hour_table, pt_wide)


def kernel(hour_idx, phone_idx, hour_table, phone_table):
    out_flat = _context_embedding_sc(
        hour_idx.astype(jnp.int32),
        phone_idx.astype(jnp.int32),
        hour_table.reshape(-1),
        phone_table.reshape(_PHONE_VOCAB // 8, 128),
    )
    return out_flat.reshape(_BATCH, 2 * _EMBED)


# speedup vs baseline: 21.3754x; 1.0308x over previous
import functools

import jax
import jax.numpy as jnp
from jax import lax
from jax.experimental import pallas as pl
from jax.experimental.pallas import tpu as pltpu
from jax.experimental.pallas import tpu_sc as plsc

_BATCH = 16384
_EMBED = 16
_HOUR_VOCAB = 24
_PHONE_VOCAB = 100000
_NC = 2
_NS = 16
_NW = _NC * _NS
_B_PER_W = _BATCH // _NW
_G = 16


@jax.jit
def _context_embedding_sc(hour_idx, phone_idx, hour_table, pt_wide):
    mesh = plsc.VectorSubcoreMesh(core_axis_name="c", subcore_axis_name="s")

    @functools.partial(
        pl.kernel,
        mesh=mesh,
        out_type=jax.ShapeDtypeStruct((_BATCH, 2 * _EMBED), jnp.float32),
        scratch_types=[
            pltpu.VMEM((_HOUR_VOCAB, _EMBED), jnp.float32),
            pltpu.VMEM((_B_PER_W,), jnp.int32),
            pltpu.VMEM((_B_PER_W,), jnp.int32),
            pltpu.VMEM((_B_PER_W,), jnp.int32),
            pltpu.VMEM((_B_PER_W, 128), jnp.float32),
            pltpu.VMEM((_B_PER_W // 2, 2 * _EMBED), jnp.float32),
            pltpu.SemaphoreType.DMA,
        ],
    )
    def k(hi_hbm, pi_hbm, ht_hbm, pt_hbm, out_hbm,
          ht_v, hi_v, pi_v, psup_v, prows_v, cat_v, sem):
        wid = lax.axis_index("s") * _NC + lax.axis_index("c")
        base = wid * _B_PER_W
        pltpu.sync_copy(hi_hbm.at[pl.ds(base, _B_PER_W)], hi_v)
        pltpu.sync_copy(pi_hbm.at[pl.ds(base, _B_PER_W)], pi_v)
        pltpu.sync_copy(ht_hbm, ht_v)

        @pl.loop(0, _B_PER_W // _G)
        def _(g):
            psup_v.at[pl.ds(g * _G, _G)][...] = (
                pi_v[pl.ds(g * _G, _G)] >> 3)

        gp = pltpu.async_copy(pt_hbm.at[psup_v], prows_v, sem)
        gp.wait()

        half = _B_PER_W // 2
        for s in range(2):
            @pl.loop(0, half // _G)
            def _(g):
                hvec = hi_v[pl.ds(s * half + g * _G, _G)]
                pvec = (pi_v[pl.ds(s * half + g * _G, _G)] & 7) * _EMBED
                for j in range(_G):
                    i = g * _G + j
                    cat_v.at[i, pl.ds(0, _EMBED)][...] = (
                        ht_v.at[hvec[j], pl.ds(0, _EMBED)][...])
                    cat_v.at[i, pl.ds(_EMBED, _EMBED)][...] = (
                        prows_v.at[s * half + i, pl.ds(pvec[j], _EMBED)][...])

            pltpu.sync_copy(cat_v, out_hbm.at[pl.ds(base + s * half, half)])

    return k(hour_idx, phone_idx, hour_table, pt_wide)


def kernel(hour_idx, phone_idx, hour_table, phone_table):
    return _context_embedding_sc(
        hour_idx.astype(jnp.int32),
        phone_idx.astype(jnp.int32),
        hour_table,
        phone_table.reshape(_PHONE_VOCAB // 8, 128),
    )
